# Initial kernel scaffold; baseline (speedup 1.0000x reference)
import jax
import jax.numpy as jnp
from jax.experimental import pallas as pl


def kernel(x, batch, W1, b1, p1, W2, b2, p2):
    raise NotImplementedError("write your pallas kernel here")


# speedup vs baseline: 5.0943x; 5.0943x over previous
import math

import jax
import jax.numpy as jnp
from jax.experimental import pallas as pl
from jax.experimental.pallas import tpu as pltpu

B_, M_, K_ = 8, 2048, 8
K1_ = int(math.ceil(0.8 * M_))
K2_ = int(math.ceil(0.5 * K1_))
CH = 512
NCH = M_ // CH


def _stage1(xrow_ref, xcol_ref, w1_ref, b1_ref, p1_ref, n1_ref,
            g_ref, mcol_ref, brow_ref, h_s, s_s):
    xr = xrow_ref[0]
    d2r = xr * xr
    col = jax.lax.broadcasted_iota(jnp.int32, (CH, M_), 1)
    row0 = jax.lax.broadcasted_iota(jnp.int32, (CH, M_), 0)
    w0 = w1_ref[0:1, :]
    w1r = w1_ref[1:2, :]
    b1 = b1_ref[...]

    def chunk_body(c, carry):
        base = c * CH
        xc = xcol_ref[0, pl.ds(base, CH), :]
        d2c = xc * xc
        dist = (d2c + d2r) - 2.0 * (xc * xr)
        dist = dist + jnp.where(col == row0 + base, 1e10, 0.0)
        acc = jnp.zeros((CH, 64), jnp.float32)
        for _ in range(K_):
            m = jnp.min(dist, axis=1, keepdims=True)
            am = jnp.min(jnp.where(dist == m, col, M_), axis=1, keepdims=True)
            sel = col == am
            xj = jnp.sum(jnp.where(sel, xr, 0.0), axis=1, keepdims=True)
            pre = (xc * w0 + (xj - xc) * w1r) + b1
            acc = acc + jax.nn.relu(pre)
            dist = jnp.where(sel, 1e30, dist)
        h = jax.nn.relu(acc * 0.125)
        s = jnp.tanh(jnp.dot(h, p1_ref[...],
                             preferred_element_type=jnp.float32,
                         precision=jax.lax.Precision.HIGHEST) / n1_ref[...])
        h_s[pl.ds(base, CH), :] = h
        s_s[pl.ds(base, CH), :] = s
        return carry

    jax.lax.fori_loop(0, NCH, chunk_body, 0)

    srow = jax.lax.transpose(s_s[...], (1, 0))
    for c in range(NCH):
        base = c * CH
        si = s_s[pl.ds(base, CH), :]
        gt = jnp.sum((srow > si).astype(jnp.int32), axis=1, keepdims=True)
        eqlt = jnp.sum(((srow == si) & (col < row0 + base)).astype(jnp.int32),
                       axis=1, keepdims=True)
        keep = (gt + eqlt) < K1_
        g_ref[0, pl.ds(base, CH), :] = (
            h_s[pl.ds(base, CH), :] * jnp.where(keep, si, 0.0))
        mcol_ref[0, pl.ds(base, CH), :] = keep.astype(jnp.float32)
    brow_ref[0] = jnp.where(
        jax.lax.transpose(mcol_ref[0], (1, 0)) > 0.0, 0.0, 1e30)


def _stage2(g_ref, mcol_ref, brow_ref, w2a_ref, w2b_ref, b2_ref, p2_ref,
            n2_ref, out_ref, gt_s, h2_s, s2_s):
    g0 = g_ref[0]
    gt_s[...] = jax.lax.transpose(g0, (1, 0))
    gT = gt_s[...]
    d2r = jnp.sum(gT * gT, axis=0, keepdims=True)
    brow = brow_ref[0]
    col = jax.lax.broadcasted_iota(jnp.int32, (CH, M_), 1)
    row0 = jax.lax.broadcasted_iota(jnp.int32, (CH, M_), 0)
    w2a = w2a_ref[...]
    w2b = w2b_ref[...]
    b2 = b2_ref[...]

    def chunk_body(c, carry):
        base = c * CH
        gch = g_ref[0, pl.ds(base, CH), :]
        d2c = jnp.sum(gch * gch, axis=1, keepdims=True)
        dist = (d2c + d2r) - 2.0 * jnp.dot(
            gch, gT, preferred_element_type=jnp.float32,
                         precision=jax.lax.Precision.HIGHEST)
        dist = dist + jnp.where(col == row0 + base, 1e10, 0.0)
        dist = dist + brow
        uch = jnp.dot(gch, w2a, preferred_element_type=jnp.float32,
                         precision=jax.lax.Precision.HIGHEST)
        acc = jnp.zeros((CH, 128), jnp.float32)
        for _ in range(K_):
            m = jnp.min(dist, axis=1, keepdims=True)
            am = jnp.min(jnp.where(dist == m, col, M_), axis=1, keepdims=True)
            sel = col == am
            gj = jnp.dot(sel.astype(jnp.float32), g0,
                         preferred_element_type=jnp.float32,
                         precision=jax.lax.Precision.HIGHEST)
            pre = (uch + jnp.dot(gj - gch, w2b,
                                 preferred_element_type=jnp.float32,
                         precision=jax.lax.Precision.HIGHEST)) + b2
            acc = acc + jax.nn.relu(pre)
            dist = jnp.where(sel, 1e30, dist)
        h2 = jax.nn.relu(acc * 0.125)
        s2 = jnp.tanh(jnp.dot(h2, p2_ref[...],
                              preferred_element_type=jnp.float32,
                         precision=jax.lax.Precision.HIGHEST) / n2_ref[...])
        mch = mcol_ref[0, pl.ds(base, CH), :]
        h2_s[pl.ds(base, CH), :] = h2
        s2_s[pl.ds(base, CH), :] = jnp.where(mch > 0.0, s2, -2.0)
        return carry

    jax.lax.fori_loop(0, NCH, chunk_body, 0)

    s2row = jax.lax.transpose(s2_s[...], (1, 0))
    out_acc = jnp.zeros((1, 128), jnp.float32)
    for c in range(NCH):
        base = c * CH
        si = s2_s[pl.ds(base, CH), :]
        gt = jnp.sum((s2row > si).astype(jnp.int32), axis=1, keepdims=True)
        eqlt = jnp.sum(((s2row == si) & (col < row0 + base)).astype(jnp.int32),
                       axis=1, keepdims=True)
        keep = (gt + eqlt) < K2_
        contrib = h2_s[pl.ds(base, CH), :] * jnp.where(keep, si, 0.0)
        out_acc = out_acc + jnp.sum(contrib, axis=0, keepdims=True)
    out_ref[0] = out_acc / jnp.float32(K2_)


def kernel(x, batch, W1, b1, p1, W2, b2, p2):
    del batch
    xb = x.reshape(B_, M_).astype(jnp.float32)
    xrow = xb.reshape(B_, 1, M_)
    xcol = xb.reshape(B_, M_, 1)
    n1 = jnp.linalg.norm(p1).reshape(1, 1)
    n2 = jnp.linalg.norm(p2).reshape(1, 1)
    p1c = p1.reshape(64, 1)
    p2c = p2.reshape(128, 1)
    b1r = b1.reshape(1, 64)
    b2r = b2.reshape(1, 128)
    w2a = W2[:64, :]
    w2b = W2[64:, :]

    g, mcol, brow = pl.pallas_call(
        _stage1,
        grid=(B_,),
        in_specs=[
            pl.BlockSpec((1, 1, M_), lambda b: (b, 0, 0)),
            pl.BlockSpec((1, M_, 1), lambda b: (b, 0, 0)),
            pl.BlockSpec((2, 64), lambda b: (0, 0)),
            pl.BlockSpec((1, 64), lambda b: (0, 0)),
            pl.BlockSpec((64, 1), lambda b: (0, 0)),
            pl.BlockSpec((1, 1), lambda b: (0, 0)),
        ],
        out_specs=[
            pl.BlockSpec((1, M_, 64), lambda b: (b, 0, 0)),
            pl.BlockSpec((1, M_, 1), lambda b: (b, 0, 0)),
            pl.BlockSpec((1, 1, M_), lambda b: (b, 0, 0)),
        ],
        out_shape=[
            jax.ShapeDtypeStruct((B_, M_, 64), jnp.float32),
            jax.ShapeDtypeStruct((B_, M_, 1), jnp.float32),
            jax.ShapeDtypeStruct((B_, 1, M_), jnp.float32),
        ],
        scratch_shapes=[
            pltpu.VMEM((M_, 64), jnp.float32),
            pltpu.VMEM((M_, 1), jnp.float32),
        ],
    )(xrow, xcol, W1, b1r, p1c, n1)

    out = pl.pallas_call(
        _stage2,
        grid=(B_,),
        in_specs=[
            pl.BlockSpec((1, M_, 64), lambda b: (b, 0, 0)),
            pl.BlockSpec((1, M_, 1), lambda b: (b, 0, 0)),
            pl.BlockSpec((1, 1, M_), lambda b: (b, 0, 0)),
            pl.BlockSpec((64, 128), lambda b: (0, 0)),
            pl.BlockSpec((64, 128), lambda b: (0, 0)),
            pl.BlockSpec((1, 128), lambda b: (0, 0)),
            pl.BlockSpec((128, 1), lambda b: (0, 0)),
            pl.BlockSpec((1, 1), lambda b: (0, 0)),
        ],
        out_specs=pl.BlockSpec((1, 1, 128), lambda b: (b, 0, 0)),
        out_shape=jax.ShapeDtypeStruct((B_, 1, 128), jnp.float32),
        scratch_shapes=[
            pltpu.VMEM((64, M_), jnp.float32),
            pltpu.VMEM((M_, 128), jnp.float32),
            pltpu.VMEM((M_, 1), jnp.float32),
        ],
    )(g, mcol, brow, w2a, w2b, b2r, p2c, n2)

    return out.reshape(B_, 128)
